# Initial kernel scaffold; baseline (speedup 1.0000x reference)
import jax
import jax.numpy as jnp
from jax.experimental import pallas as pl


def kernel(x, pos, batch, edge_index, W_embed, b_embed, W_msg, W_rbf, W_upd, b_upd, W_graph, W1, b1, W2, b2, W3, b3, W4, b4, W_out):
    raise NotImplementedError("write your pallas kernel here")


# speedup vs baseline: 2.4099x; 2.4099x over previous
import functools

import jax
import jax.numpy as jnp
from jax import lax
from jax.experimental import pallas as pl
from jax.experimental.pallas import tpu as pltpu
from jax.experimental.pallas import tpu_sc as plsc

N = 50000
E = 800000
D_IN = 128
D_NODE = 128
D_MSG = 64
HALF = 32
N_RADIAL = 32
D_GRAPH = 256
FC = 256
D_OUT = 128
NG = 100
NGP = 128
CUTOFF = 6.0

NC = 2
NS = 16
L = 16

C = 128

f32 = jnp.float32
i32 = jnp.int32


NB1 = 2000
G1 = N // NB1


def _k1_body(x_ref, we_ref, be_ref, wm_ref, h0_ref, m_ref):
    h = jax.nn.gelu(jnp.dot(x_ref[...], we_ref[...],
                            preferred_element_type=f32) + be_ref[...])
    h0_ref[...] = h
    mm = jnp.dot(h, wm_ref[...], preferred_element_type=f32)
    m_ref[...] = jnp.stack([mm[:, :HALF], mm[:, HALF:]])


def _k1(x, W_embed, b_embed, W_msg):
    return pl.pallas_call(
        _k1_body,
        grid=(G1,),
        in_specs=[
            pl.BlockSpec((NB1, D_IN), lambda g: (g, 0)),
            pl.BlockSpec((D_IN, D_NODE), lambda g: (0, 0)),
            pl.BlockSpec((1, D_NODE), lambda g: (0, 0)),
            pl.BlockSpec((D_NODE, D_MSG), lambda g: (0, 0)),
        ],
        out_specs=[
            pl.BlockSpec((NB1, D_NODE), lambda g: (g, 0)),
            pl.BlockSpec((2, NB1, HALF), lambda g: (0, g, 0)),
        ],
        out_shape=[
            jax.ShapeDtypeStruct((N, D_NODE), f32),
            jax.ShapeDtypeStruct((2, N, HALF), f32),
        ],
    )(x, W_embed, b_embed, W_msg)


EW2 = E // (NC * NS)
N2 = (EW2 + C - 1) // C


def _k2_body(pos_hbm, src_hbm, dst_hbm, diff_hbm, idx_s, idx_d, ps, pd, sem):
    wid = lax.axis_index("s") * NC + lax.axis_index("c")
    base0 = wid * EW2

    def chunk(ci, _):
        base = base0 + jnp.minimum(ci * C, EW2 - C)
        pltpu.sync_copy(src_hbm.at[pl.ds(base, C)], idx_s)
        pltpu.sync_copy(dst_hbm.at[pl.ds(base, C)], idx_d)
        pltpu.async_copy(pos_hbm.at[idx_s], ps, sem).wait()
        pltpu.async_copy(pos_hbm.at[idx_d], pd, sem).wait()

        def sub_row(j, _):
            ps[j, :] = ps[j, :] - pd[j, :]
            return 0
        lax.fori_loop(0, C, sub_row, 0)
        pltpu.sync_copy(ps, diff_hbm.at[pl.ds(base, C)])
        return 0

    lax.fori_loop(0, N2, chunk, 0)


def _k2(pos16, src_i, dst_i):
    mesh = plsc.VectorSubcoreMesh(core_axis_name="c", subcore_axis_name="s")
    return pl.kernel(
        _k2_body,
        out_type=jax.ShapeDtypeStruct((E, L), f32),
        mesh=mesh,
        scratch_types=[
            pltpu.VMEM((C,), i32),
            pltpu.VMEM((C,), i32),
            pltpu.VMEM((C, L), f32),
            pltpu.VMEM((C, L), f32),
            pltpu.SemaphoreType.DMA,
        ],
        compiler_params=pltpu.CompilerParams(use_tc_tiling_on_sc=False),
    )(pos16, src_i, dst_i)


EB3 = 8000
G3 = E // EB3


def _k3_body(diff_ref, wr_ref, ew_ref):
    t4 = diff_ref[...]
    ssq = jnp.sum(t4 * t4, axis=1, keepdims=True)
    d = jnp.sqrt(ssq + 1e-8)
    mu = lax.broadcasted_iota(i32, (1, N_RADIAL), 1).astype(f32) * (
        CUTOFF / (N_RADIAL - 1))
    sigma = CUTOFF / N_RADIAL
    t = jnp.broadcast_to(d, (EB3, N_RADIAL)) - mu
    rbf = jnp.exp(t * t * (-1.0 / (2.0 * sigma * sigma)))
    wr = wr_ref[...]
    ew0 = jnp.dot(rbf, wr[:, :HALF], preferred_element_type=f32)
    ew1 = jnp.dot(rbf, wr[:, HALF:], preferred_element_type=f32)
    ew_ref[...] = jnp.stack([ew0, ew1])


def _k3(diff, W_rbf):
    return pl.pallas_call(
        _k3_body,
        grid=(G3,),
        in_specs=[
            pl.BlockSpec((EB3, L), lambda g: (g, 0)),
            pl.BlockSpec((N_RADIAL, D_MSG), lambda g: (0, 0)),
        ],
        out_specs=pl.BlockSpec((2, EB3, HALF), lambda g: (0, g, 0)),
        out_shape=jax.ShapeDtypeStruct((2, E, HALF), f32),
    )(diff, W_rbf)


EW4 = E // NS
N4 = EW4 // C
TAIL4 = EW4 - N4 * C
NROW = N // NS


def _k4_body(m_hbm, ew_hbm, src_hbm, dst_hbm, zeros_hbm, agg_hbm,
             acc, idx_s, idx_d, rows, ewb, sem):
    core = lax.axis_index("c")
    sid = lax.axis_index("s")

    pltpu.sync_copy(zeros_hbm.at[pl.ds(sid * NROW, NROW)],
                    acc.at[pl.ds(sid * NROW, NROW)])
    plsc.subcore_barrier()

    def do_chunk(base, nvalid):
        nload = ((nvalid + 7) // 8) * 8
        pltpu.sync_copy(src_hbm.at[pl.ds(base, nload)],
                        idx_s.at[pl.ds(0, nload)])
        pltpu.sync_copy(dst_hbm.at[pl.ds(base, nload)],
                        idx_d.at[pl.ds(0, nload)])
        pltpu.async_copy(m_hbm.at[core].at[idx_s], rows, sem).wait()
        pltpu.sync_copy(ew_hbm.at[core].at[pl.ds(base, nload)],
                        ewb.at[pl.ds(0, nload)])

        def mul_row(j, _):
            for k in range(HALF // L):
                sl = pl.ds(k * L, L)
                rows[j, sl] = rows[j, sl] * ewb[j, sl]
            return 0
        lax.fori_loop(0, nvalid, mul_row, 0)
        if nvalid < C:
            def zero_row(j, _):
                for k in range(HALF // L):
                    rows[j, pl.ds(k * L, L)] = jnp.zeros((L,), f32)
                return 0
            lax.fori_loop(nvalid, C, zero_row, 0)
        pltpu.sync_copy(rows, acc.at[idx_d], add=True)

    base0 = sid * EW4

    def chunk(ci, _):
        do_chunk(base0 + ci * C, C)
        return 0

    lax.fori_loop(0, N4, chunk, 0)
    if TAIL4:
        do_chunk(base0 + N4 * C, TAIL4)
    plsc.subcore_barrier()

    pltpu.sync_copy(acc.at[pl.ds(sid * NROW, NROW)],
                    agg_hbm.at[core].at[pl.ds(sid * NROW, NROW)])


def _k4(m_pk, ew_pk, src_i, dst_i, zeros_h):
    mesh = plsc.VectorSubcoreMesh(core_axis_name="c", subcore_axis_name="s")
    return pl.kernel(
        _k4_body,
        out_type=jax.ShapeDtypeStruct((2, N, HALF), f32),
        mesh=mesh,
        scratch_types=[
            pltpu.VMEM_SHARED((N, HALF), f32),
            pltpu.VMEM((C,), i32),
            pltpu.VMEM((C,), i32),
            pltpu.VMEM((C, HALF), f32),
            pltpu.VMEM((C, HALF), f32),
            pltpu.SemaphoreType.DMA,
        ],
        compiler_params=pltpu.CompilerParams(use_tc_tiling_on_sc=False),
    )(m_pk, ew_pk, src_i, dst_i, zeros_h)


NB5 = 2000
G5 = N // NB5


def _k5_body(h0_ref, agg_ref, b_ref, wu_ref, bu_ref, wg_ref,
             w1_ref, b1_ref, w2_ref, b2_ref, w3_ref, b3_ref,
             w4_ref, b4_ref, wo_ref, out_ref, pooled, counts):
    g = pl.program_id(0)

    a = jnp.concatenate([agg_ref[0], agg_ref[1]], axis=1)
    upd = jnp.dot(a, wu_ref[...], preferred_element_type=f32) + bu_ref[...]
    h = h0_ref[...] + jax.nn.gelu(upd)
    gg = jnp.dot(h, wg_ref[...], preferred_element_type=f32)

    bb = jnp.broadcast_to(b_ref[0], (NGP, NB5))
    gid = lax.broadcasted_iota(i32, (NGP, NB5), 0).astype(f32)
    oh = jnp.where(bb == gid, 1.0, 0.0).astype(f32)

    @pl.when(g == 0)
    def _():
        pooled[...] = jnp.zeros((NGP, D_GRAPH), f32)
        counts[...] = jnp.zeros((NGP, NGP), f32)

    pooled[...] += jnp.dot(oh, gg, preferred_element_type=f32)
    counts[...] += jnp.broadcast_to(
        jnp.sum(oh, axis=1, keepdims=True), (NGP, NGP))

    @pl.when(g == G5 - 1)
    def _():
        cnt = jnp.maximum(counts[...][:, 0:1], 1.0)
        pm = pooled[...] / cnt
        y = jax.nn.gelu(jnp.dot(pm, w1_ref[...],
                                preferred_element_type=f32) + b1_ref[...])
        y = jax.nn.gelu(jnp.dot(y, w2_ref[...],
                                preferred_element_type=f32) + b2_ref[...])
        y = jax.nn.gelu(jnp.dot(y, w3_ref[...],
                                preferred_element_type=f32) + b3_ref[...])
        y = jax.nn.gelu(jnp.dot(y, w4_ref[...],
                                preferred_element_type=f32) + b4_ref[...])
        out_ref[...] = jnp.dot(y, wo_ref[...], preferred_element_type=f32)


def _k5(h0, agg_pk, batchf3, W_upd, b_upd, W_graph,
        W1, b1, W2, b2, W3, b3, W4, b4, W_out):
    full = lambda shape: pl.BlockSpec(shape, lambda g: tuple(0 for _ in shape))
    return pl.pallas_call(
        _k5_body,
        grid=(G5,),
        in_specs=[
            pl.BlockSpec((NB5, D_NODE), lambda g: (g, 0)),
            pl.BlockSpec((2, NB5, HALF), lambda g: (0, g, 0)),
            pl.BlockSpec((1, 1, NB5), lambda g: (g, 0, 0)),
            full((D_MSG, D_NODE)),
            full((1, D_NODE)),
            full((D_NODE, D_GRAPH)),
            full((D_GRAPH, FC)),
            full((1, FC)),
            full((FC, FC)),
            full((1, FC)),
            full((FC, FC)),
            full((1, FC)),
            full((FC, FC)),
            full((1, FC)),
            full((FC, D_OUT)),
        ],
        out_specs=pl.BlockSpec((NGP, D_OUT), lambda g: (0, 0)),
        out_shape=jax.ShapeDtypeStruct((NGP, D_OUT), f32),
        scratch_shapes=[
            pltpu.VMEM((NGP, D_GRAPH), f32),
            pltpu.VMEM((NGP, NGP), f32),
        ],
    )(h0, agg_pk, batchf3, W_upd, b_upd, W_graph,
      W1, b1, W2, b2, W3, b3, W4, b4, W_out)


def kernel(x, pos, batch, edge_index, W_embed, b_embed, W_msg, W_rbf,
           W_upd, b_upd, W_graph, W1, b1, W2, b2, W3, b3, W4, b4, W_out):
    src_i = edge_index[0].astype(i32)
    dst_i = edge_index[1].astype(i32)
    pos16 = jnp.concatenate([pos, jnp.zeros((N, L - 3), f32)], axis=1)
    batchf3 = batch.astype(f32).reshape(G5, 1, NB5)
    zeros_h = jnp.zeros((N, HALF), f32)

    h0, m_pk = _k1(x, W_embed, b_embed.reshape(1, D_NODE), W_msg)
    diff = _k2(pos16, src_i, dst_i)
    ew_pk = _k3(diff, W_rbf)
    agg_pk = _k4(m_pk, ew_pk, src_i, dst_i, zeros_h)
    out_full = _k5(h0, agg_pk, batchf3, W_upd,
                   b_upd.reshape(1, D_NODE), W_graph,
                   W1, b1.reshape(1, FC), W2, b2.reshape(1, FC),
                   W3, b3.reshape(1, FC), W4, b4.reshape(1, FC), W_out)
    return out_full[:NG]


# speedup vs baseline: 2.9493x; 1.2238x over previous
import functools

import jax
import jax.numpy as jnp
from jax import lax
from jax.experimental import pallas as pl
from jax.experimental.pallas import tpu as pltpu
from jax.experimental.pallas import tpu_sc as plsc

N = 50000
E = 800000
D_IN = 128
D_NODE = 128
D_MSG = 64
HALF = 32
N_RADIAL = 32
D_GRAPH = 256
FC = 256
D_OUT = 128
NG = 100
NGP = 128
CUTOFF = 6.0

NC = 2
NS = 16
L = 16

C = 128

f32 = jnp.float32
i32 = jnp.int32


NB1 = 2000
G1 = N // NB1


def _k1_body(x_ref, we_ref, be_ref, wm_ref, h0_ref, m_ref):
    h = jax.nn.gelu(jnp.dot(x_ref[...], we_ref[...],
                            preferred_element_type=f32) + be_ref[...])
    h0_ref[...] = h
    mm = jnp.dot(h, wm_ref[...], preferred_element_type=f32)
    m_ref[...] = jnp.stack([mm[:, :HALF], mm[:, HALF:]])


def _k1(x, W_embed, b_embed, W_msg):
    return pl.pallas_call(
        _k1_body,
        grid=(G1,),
        in_specs=[
            pl.BlockSpec((NB1, D_IN), lambda g: (g, 0)),
            pl.BlockSpec((D_IN, D_NODE), lambda g: (0, 0)),
            pl.BlockSpec((1, D_NODE), lambda g: (0, 0)),
            pl.BlockSpec((D_NODE, D_MSG), lambda g: (0, 0)),
        ],
        out_specs=[
            pl.BlockSpec((NB1, D_NODE), lambda g: (g, 0)),
            pl.BlockSpec((2, NB1, HALF), lambda g: (0, g, 0)),
        ],
        out_shape=[
            jax.ShapeDtypeStruct((N, D_NODE), f32),
            jax.ShapeDtypeStruct((2, N, HALF), f32),
        ],
    )(x, W_embed, b_embed, W_msg)


EW2 = E // (NC * NS)
N2 = (EW2 + C - 1) // C


def _k2_body(pos_hbm, src_hbm, dst_hbm, diff_hbm, idx_s, idx_d, ps, pd,
             sem, sem2):
    wid = lax.axis_index("s") * NC + lax.axis_index("c")
    base0 = wid * EW2

    def chunk(ci, _):
        base = base0 + jnp.minimum(ci * C, EW2 - C)
        pltpu.sync_copy(src_hbm.at[pl.ds(base, C)], idx_s)
        pltpu.sync_copy(dst_hbm.at[pl.ds(base, C)], idx_d)
        cps = pltpu.async_copy(pos_hbm.at[idx_s], ps, sem)
        cpd = pltpu.async_copy(pos_hbm.at[idx_d], pd, sem2)
        cps.wait()
        cpd.wait()

        def sub8(i, _):
            for u in range(8):
                j = i * 8 + u
                ps[j, :] = ps[j, :] - pd[j, :]
            return 0
        lax.fori_loop(0, C // 8, sub8, 0)
        pltpu.sync_copy(ps, diff_hbm.at[pl.ds(base, C)])
        return 0

    lax.fori_loop(0, N2, chunk, 0)


def _k2(pos16, src_i, dst_i):
    mesh = plsc.VectorSubcoreMesh(core_axis_name="c", subcore_axis_name="s")
    return pl.kernel(
        _k2_body,
        out_type=jax.ShapeDtypeStruct((E, L), f32),
        mesh=mesh,
        scratch_types=[
            pltpu.VMEM((C,), i32),
            pltpu.VMEM((C,), i32),
            pltpu.VMEM((C, L), f32),
            pltpu.VMEM((C, L), f32),
            pltpu.SemaphoreType.DMA,
            pltpu.SemaphoreType.DMA,
        ],
        compiler_params=pltpu.CompilerParams(use_tc_tiling_on_sc=False),
    )(pos16, src_i, dst_i)


EB3 = 8000
G3 = E // EB3


def _k3_body(diff_ref, wr_ref, ew_ref):
    t4 = diff_ref[...]
    ssq = jnp.sum(t4 * t4, axis=1, keepdims=True)
    d = jnp.sqrt(ssq + 1e-8)
    mu = lax.broadcasted_iota(i32, (1, N_RADIAL), 1).astype(f32) * (
        CUTOFF / (N_RADIAL - 1))
    sigma = CUTOFF / N_RADIAL
    t = jnp.broadcast_to(d, (EB3, N_RADIAL)) - mu
    rbf = jnp.exp(t * t * (-1.0 / (2.0 * sigma * sigma)))
    wr = wr_ref[...]
    ew0 = jnp.dot(rbf, wr[:, :HALF], preferred_element_type=f32)
    ew1 = jnp.dot(rbf, wr[:, HALF:], preferred_element_type=f32)
    ew_ref[...] = jnp.stack([ew0, ew1])


def _k3(diff, W_rbf):
    return pl.pallas_call(
        _k3_body,
        grid=(G3,),
        in_specs=[
            pl.BlockSpec((EB3, L), lambda g: (g, 0)),
            pl.BlockSpec((N_RADIAL, D_MSG), lambda g: (0, 0)),
        ],
        out_specs=pl.BlockSpec((2, EB3, HALF), lambda g: (0, g, 0)),
        out_shape=jax.ShapeDtypeStruct((2, E, HALF), f32),
    )(diff, W_rbf)


EW4 = E // NS
N4 = EW4 // C
TAIL4 = EW4 - N4 * C
NROW = N // NS


def _k4_body(m_hbm, ew_hbm, src_hbm, dst_hbm, zeros_hbm, agg_hbm,
             acc, idx_s, idx_d, rows, ewb, sem):
    core = lax.axis_index("c")
    sid = lax.axis_index("s")

    pltpu.sync_copy(zeros_hbm.at[pl.ds(sid * NROW, NROW)],
                    acc.at[pl.ds(sid * NROW, NROW)])
    plsc.subcore_barrier()

    def do_chunk(base, nvalid):
        nload = ((nvalid + 7) // 8) * 8
        pltpu.sync_copy(src_hbm.at[pl.ds(base, nload)],
                        idx_s.at[pl.ds(0, nload)])
        pltpu.sync_copy(dst_hbm.at[pl.ds(base, nload)],
                        idx_d.at[pl.ds(0, nload)])
        cpm = pltpu.async_copy(m_hbm.at[core].at[idx_s], rows, sem)
        pltpu.sync_copy(ew_hbm.at[core].at[pl.ds(base, nload)],
                        ewb.at[pl.ds(0, nload)])
        cpm.wait()

        def mul8(i, _):
            for u in range(8):
                j = i * 8 + u
                for k in range(HALF // L):
                    sl = pl.ds(k * L, L)
                    rows[j, sl] = rows[j, sl] * ewb[j, sl]
            return 0
        lax.fori_loop(0, nvalid // 8, mul8, 0)
        if nvalid < C:
            def zero8(i, _):
                for u in range(8):
                    j = nvalid + i * 8 + u
                    for k in range(HALF // L):
                        rows[j, pl.ds(k * L, L)] = jnp.zeros((L,), f32)
                return 0
            lax.fori_loop(0, (C - nvalid) // 8, zero8, 0)
        pltpu.sync_copy(rows, acc.at[idx_d], add=True)

    base0 = sid * EW4

    def chunk(ci, _):
        do_chunk(base0 + ci * C, C)
        return 0

    lax.fori_loop(0, N4, chunk, 0)
    if TAIL4:
        do_chunk(base0 + N4 * C, TAIL4)
    plsc.subcore_barrier()

    pltpu.sync_copy(acc.at[pl.ds(sid * NROW, NROW)],
                    agg_hbm.at[core].at[pl.ds(sid * NROW, NROW)])


def _k4(m_pk, ew_pk, src_i, dst_i, zeros_h):
    mesh = plsc.VectorSubcoreMesh(core_axis_name="c", subcore_axis_name="s")
    return pl.kernel(
        _k4_body,
        out_type=jax.ShapeDtypeStruct((2, N, HALF), f32),
        mesh=mesh,
        scratch_types=[
            pltpu.VMEM_SHARED((N, HALF), f32),
            pltpu.VMEM((C,), i32),
            pltpu.VMEM((C,), i32),
            pltpu.VMEM((C, HALF), f32),
            pltpu.VMEM((C, HALF), f32),
            pltpu.SemaphoreType.DMA,
        ],
        compiler_params=pltpu.CompilerParams(use_tc_tiling_on_sc=False),
    )(m_pk, ew_pk, src_i, dst_i, zeros_h)


NB5 = 2000
G5 = N // NB5


def _k5_body(h0_ref, agg_ref, b_ref, wu_ref, bu_ref, wg_ref,
             w1_ref, b1_ref, w2_ref, b2_ref, w3_ref, b3_ref,
             w4_ref, b4_ref, wo_ref, out_ref, pooled, counts):
    g = pl.program_id(0)

    a = jnp.concatenate([agg_ref[0], agg_ref[1]], axis=1)
    upd = jnp.dot(a, wu_ref[...], preferred_element_type=f32) + bu_ref[...]
    h = h0_ref[...] + jax.nn.gelu(upd)
    gg = jnp.dot(h, wg_ref[...], preferred_element_type=f32)

    bb = jnp.broadcast_to(b_ref[0], (NGP, NB5))
    gid = lax.broadcasted_iota(i32, (NGP, NB5), 0).astype(f32)
    oh = jnp.where(bb == gid, 1.0, 0.0).astype(f32)

    @pl.when(g == 0)
    def _():
        pooled[...] = jnp.zeros((NGP, D_GRAPH), f32)
        counts[...] = jnp.zeros((NGP, NGP), f32)

    pooled[...] += jnp.dot(oh, gg, preferred_element_type=f32)
    counts[...] += jnp.broadcast_to(
        jnp.sum(oh, axis=1, keepdims=True), (NGP, NGP))

    @pl.when(g == G5 - 1)
    def _():
        cnt = jnp.maximum(counts[...][:, 0:1], 1.0)
        pm = pooled[...] / cnt
        y = jax.nn.gelu(jnp.dot(pm, w1_ref[...],
                                preferred_element_type=f32) + b1_ref[...])
        y = jax.nn.gelu(jnp.dot(y, w2_ref[...],
                                preferred_element_type=f32) + b2_ref[...])
        y = jax.nn.gelu(jnp.dot(y, w3_ref[...],
                                preferred_element_type=f32) + b3_ref[...])
        y = jax.nn.gelu(jnp.dot(y, w4_ref[...],
                                preferred_element_type=f32) + b4_ref[...])
        out_ref[...] = jnp.dot(y, wo_ref[...], preferred_element_type=f32)


def _k5(h0, agg_pk, batchf3, W_upd, b_upd, W_graph,
        W1, b1, W2, b2, W3, b3, W4, b4, W_out):
    full = lambda shape: pl.BlockSpec(shape, lambda g: tuple(0 for _ in shape))
    return pl.pallas_call(
        _k5_body,
        grid=(G5,),
        in_specs=[
            pl.BlockSpec((NB5, D_NODE), lambda g: (g, 0)),
            pl.BlockSpec((2, NB5, HALF), lambda g: (0, g, 0)),
            pl.BlockSpec((1, 1, NB5), lambda g: (g, 0, 0)),
            full((D_MSG, D_NODE)),
            full((1, D_NODE)),
            full((D_NODE, D_GRAPH)),
            full((D_GRAPH, FC)),
            full((1, FC)),
            full((FC, FC)),
            full((1, FC)),
            full((FC, FC)),
            full((1, FC)),
            full((FC, FC)),
            full((1, FC)),
            full((FC, D_OUT)),
        ],
        out_specs=pl.BlockSpec((NGP, D_OUT), lambda g: (0, 0)),
        out_shape=jax.ShapeDtypeStruct((NGP, D_OUT), f32),
        scratch_shapes=[
            pltpu.VMEM((NGP, D_GRAPH), f32),
            pltpu.VMEM((NGP, NGP), f32),
        ],
    )(h0, agg_pk, batchf3, W_upd, b_upd, W_graph,
      W1, b1, W2, b2, W3, b3, W4, b4, W_out)


def kernel(x, pos, batch, edge_index, W_embed, b_embed, W_msg, W_rbf,
           W_upd, b_upd, W_graph, W1, b1, W2, b2, W3, b3, W4, b4, W_out):
    src_i = edge_index[0].astype(i32)
    dst_i = edge_index[1].astype(i32)
    pos16 = jnp.concatenate([pos, jnp.zeros((N, L - 3), f32)], axis=1)
    batchf3 = batch.astype(f32).reshape(G5, 1, NB5)
    zeros_h = jnp.zeros((N, HALF), f32)

    h0, m_pk = _k1(x, W_embed, b_embed.reshape(1, D_NODE), W_msg)
    diff = _k2(pos16, src_i, dst_i)
    ew_pk = _k3(diff, W_rbf)
    agg_pk = _k4(m_pk, ew_pk, src_i, dst_i, zeros_h)
    out_full = _k5(h0, agg_pk, batchf3, W_upd,
                   b_upd.reshape(1, D_NODE), W_graph,
                   W1, b1.reshape(1, FC), W2, b2.reshape(1, FC),
                   W3, b3.reshape(1, FC), W4, b4.reshape(1, FC), W_out)
    return out_full[:NG]


# speedup vs baseline: 3.3461x; 1.1346x over previous
import functools

import jax
import jax.numpy as jnp
from jax import lax
from jax.experimental import pallas as pl
from jax.experimental.pallas import tpu as pltpu
from jax.experimental.pallas import tpu_sc as plsc

N = 50000
E = 800000
D_IN = 128
D_NODE = 128
D_MSG = 64
HALF = 32
N_RADIAL = 32
D_GRAPH = 256
FC = 256
D_OUT = 128
NG = 100
NGP = 128
CUTOFF = 6.0

NC = 2
NS = 16
L = 16

C = 128

f32 = jnp.float32
i32 = jnp.int32


NB1 = 2000
G1 = N // NB1


def _k1_body(x_ref, we_ref, be_ref, wm_ref, h0_ref, m_ref):
    h = jax.nn.gelu(jnp.dot(x_ref[...], we_ref[...],
                            preferred_element_type=f32) + be_ref[...])
    h0_ref[...] = h
    mm = jnp.dot(h, wm_ref[...], preferred_element_type=f32)
    m_ref[...] = jnp.stack([mm[:, :HALF], mm[:, HALF:]])


def _k1(x, W_embed, b_embed, W_msg):
    return pl.pallas_call(
        _k1_body,
        grid=(G1,),
        in_specs=[
            pl.BlockSpec((NB1, D_IN), lambda g: (g, 0)),
            pl.BlockSpec((D_IN, D_NODE), lambda g: (0, 0)),
            pl.BlockSpec((1, D_NODE), lambda g: (0, 0)),
            pl.BlockSpec((D_NODE, D_MSG), lambda g: (0, 0)),
        ],
        out_specs=[
            pl.BlockSpec((NB1, D_NODE), lambda g: (g, 0)),
            pl.BlockSpec((2, NB1, HALF), lambda g: (0, g, 0)),
        ],
        out_shape=[
            jax.ShapeDtypeStruct((N, D_NODE), f32),
            jax.ShapeDtypeStruct((2, N, HALF), f32),
        ],
    )(x, W_embed, b_embed, W_msg)


EW2 = E // (NC * NS)
N2 = (EW2 + C - 1) // C


def _k2_body(pos_hbm, src_hbm, dst_hbm, diff_hbm, idx_s, idx_d, ps, pd,
             sem, sem2):
    wid = lax.axis_index("s") * NC + lax.axis_index("c")
    base0 = wid * EW2

    def chunk(ci, _):
        base = base0 + jnp.minimum(ci * C, EW2 - C)
        pltpu.sync_copy(src_hbm.at[pl.ds(base, C)], idx_s)
        pltpu.sync_copy(dst_hbm.at[pl.ds(base, C)], idx_d)
        cps = pltpu.async_copy(pos_hbm.at[idx_s], ps, sem)
        cpd = pltpu.async_copy(pos_hbm.at[idx_d], pd, sem2)
        cps.wait()
        cpd.wait()

        def sub8(i, _):
            for u in range(8):
                j = i * 8 + u
                ps[j, :] = ps[j, :] - pd[j, :]
            return 0
        lax.fori_loop(0, C // 8, sub8, 0)
        pltpu.sync_copy(ps, diff_hbm.at[pl.ds(base, C)])
        return 0

    lax.fori_loop(0, N2, chunk, 0)


def _k2(pos16, src_i, dst_i):
    mesh = plsc.VectorSubcoreMesh(core_axis_name="c", subcore_axis_name="s")
    return pl.kernel(
        _k2_body,
        out_type=jax.ShapeDtypeStruct((E, L), f32),
        mesh=mesh,
        scratch_types=[
            pltpu.VMEM((C,), i32),
            pltpu.VMEM((C,), i32),
            pltpu.VMEM((C, L), f32),
            pltpu.VMEM((C, L), f32),
            pltpu.SemaphoreType.DMA,
            pltpu.SemaphoreType.DMA,
        ],
        compiler_params=pltpu.CompilerParams(use_tc_tiling_on_sc=False),
    )(pos16, src_i, dst_i)


EB3 = 8000
G3 = E // EB3


def _k3_body(diff_ref, wr_ref, ew_ref):
    t4 = diff_ref[...]
    ssq = jnp.sum(t4 * t4, axis=1, keepdims=True)
    d = jnp.sqrt(ssq + 1e-8)
    mu = lax.broadcasted_iota(i32, (1, N_RADIAL), 1).astype(f32) * (
        CUTOFF / (N_RADIAL - 1))
    sigma = CUTOFF / N_RADIAL
    t = jnp.broadcast_to(d, (EB3, N_RADIAL)) - mu
    rbf = jnp.exp(t * t * (-1.0 / (2.0 * sigma * sigma)))
    wr = wr_ref[...]
    ew0 = jnp.dot(rbf, wr[:, :HALF], preferred_element_type=f32)
    ew1 = jnp.dot(rbf, wr[:, HALF:], preferred_element_type=f32)
    ew_ref[...] = jnp.stack([ew0, ew1])


def _k3(diff, W_rbf):
    return pl.pallas_call(
        _k3_body,
        grid=(G3,),
        in_specs=[
            pl.BlockSpec((EB3, L), lambda g: (g, 0)),
            pl.BlockSpec((N_RADIAL, D_MSG), lambda g: (0, 0)),
        ],
        out_specs=pl.BlockSpec((2, EB3, HALF), lambda g: (0, g, 0)),
        out_shape=jax.ShapeDtypeStruct((2, E, HALF), f32),
    )(diff, W_rbf)


EW4 = E // NS
N4 = EW4 // C
TAIL4 = EW4 - N4 * C
NROW = N // NS


def _k4_body(m_hbm, ew_hbm, src_hbm, dst_hbm, zeros_hbm, agg_hbm,
             acc, is0, id0, rw0, eb0, is1, id1, rw1, eb1,
             sg0, se0, sg1, se1):
    core = lax.axis_index("c")
    sid = lax.axis_index("s")

    pltpu.sync_copy(zeros_hbm.at[pl.ds(sid * NROW, NROW)],
                    acc.at[pl.ds(sid * NROW, NROW)])
    plsc.subcore_barrier()

    base0 = sid * EW4
    bufs = ((is0, id0, rw0, eb0, sg0, se0),
            (is1, id1, rw1, eb1, sg1, se1))

    def fire(c, p, nload):
        isx, idx, rwx, ebx, sg, se = bufs[p]
        base = base0 + c * C
        pltpu.sync_copy(src_hbm.at[pl.ds(base, nload)],
                        isx.at[pl.ds(0, nload)])
        pltpu.sync_copy(dst_hbm.at[pl.ds(base, nload)],
                        idx.at[pl.ds(0, nload)])
        pltpu.async_copy(m_hbm.at[core].at[isx], rwx, sg)
        pltpu.async_copy(ew_hbm.at[core].at[pl.ds(base, nload)],
                        ebx.at[pl.ds(0, nload)], se)

    def process(p, nvalid):
        isx, idx, rwx, ebx, sg, se = bufs[p]
        pltpu.make_async_copy(m_hbm.at[core].at[isx], rwx, sg).wait()
        pltpu.make_async_copy(ew_hbm.at[core].at[pl.ds(0, nvalid)],
                              ebx.at[pl.ds(0, nvalid)], se).wait()

        def mul8(i, _):
            for u in range(8):
                j = i * 8 + u
                for k in range(HALF // L):
                    sl = pl.ds(k * L, L)
                    rwx[j, sl] = rwx[j, sl] * ebx[j, sl]
            return 0
        lax.fori_loop(0, nvalid // 8, mul8, 0)
        if nvalid < C:
            def zero8(i, _):
                for u in range(8):
                    j = nvalid + i * 8 + u
                    for k in range(HALF // L):
                        rwx[j, pl.ds(k * L, L)] = jnp.zeros((L,), f32)
                return 0
            lax.fori_loop(0, (C - nvalid) // 8, zero8, 0)
        pltpu.sync_copy(rwx, acc.at[idx], add=True)

    fire(0, 0, C)
    fire(1, 1, C)

    def body2(i2, _):
        i = i2 * 2
        process(0, C)
        fire(i + 2, 0, C)
        process(1, C)
        fire(i + 3, 1, C)
        return 0
    lax.fori_loop(0, (N4 - 2) // 2, body2, 0)

    process(0, C)
    fire(N4, 0, TAIL4)
    process(1, C)
    process(0, TAIL4)
    plsc.subcore_barrier()

    pltpu.sync_copy(acc.at[pl.ds(sid * NROW, NROW)],
                    agg_hbm.at[core].at[pl.ds(sid * NROW, NROW)])


def _k4(m_pk, ew_pk, src_i, dst_i, zeros_h):
    mesh = plsc.VectorSubcoreMesh(core_axis_name="c", subcore_axis_name="s")
    return pl.kernel(
        _k4_body,
        out_type=jax.ShapeDtypeStruct((2, N, HALF), f32),
        mesh=mesh,
        scratch_types=[
            pltpu.VMEM_SHARED((N, HALF), f32),
            pltpu.VMEM((C,), i32),
            pltpu.VMEM((C,), i32),
            pltpu.VMEM((C, HALF), f32),
            pltpu.VMEM((C, HALF), f32),
            pltpu.VMEM((C,), i32),
            pltpu.VMEM((C,), i32),
            pltpu.VMEM((C, HALF), f32),
            pltpu.VMEM((C, HALF), f32),
            pltpu.SemaphoreType.DMA,
            pltpu.SemaphoreType.DMA,
            pltpu.SemaphoreType.DMA,
            pltpu.SemaphoreType.DMA,
        ],
        compiler_params=pltpu.CompilerParams(use_tc_tiling_on_sc=False),
    )(m_pk, ew_pk, src_i, dst_i, zeros_h)


NB5 = 2000
G5 = N // NB5


def _k5_body(h0_ref, agg_ref, b_ref, wu_ref, bu_ref, wg_ref,
             w1_ref, b1_ref, w2_ref, b2_ref, w3_ref, b3_ref,
             w4_ref, b4_ref, wo_ref, out_ref, pooled, counts):
    g = pl.program_id(0)

    a = jnp.concatenate([agg_ref[0], agg_ref[1]], axis=1)
    upd = jnp.dot(a, wu_ref[...], preferred_element_type=f32) + bu_ref[...]
    h = h0_ref[...] + jax.nn.gelu(upd)
    gg = jnp.dot(h, wg_ref[...], preferred_element_type=f32)

    bb = jnp.broadcast_to(b_ref[0], (NGP, NB5))
    gid = lax.broadcasted_iota(i32, (NGP, NB5), 0).astype(f32)
    oh = jnp.where(bb == gid, 1.0, 0.0).astype(f32)

    @pl.when(g == 0)
    def _():
        pooled[...] = jnp.zeros((NGP, D_GRAPH), f32)
        counts[...] = jnp.zeros((NGP, NGP), f32)

    pooled[...] += jnp.dot(oh, gg, preferred_element_type=f32)
    counts[...] += jnp.broadcast_to(
        jnp.sum(oh, axis=1, keepdims=True), (NGP, NGP))

    @pl.when(g == G5 - 1)
    def _():
        cnt = jnp.maximum(counts[...][:, 0:1], 1.0)
        pm = pooled[...] / cnt
        y = jax.nn.gelu(jnp.dot(pm, w1_ref[...],
                                preferred_element_type=f32) + b1_ref[...])
        y = jax.nn.gelu(jnp.dot(y, w2_ref[...],
                                preferred_element_type=f32) + b2_ref[...])
        y = jax.nn.gelu(jnp.dot(y, w3_ref[...],
                                preferred_element_type=f32) + b3_ref[...])
        y = jax.nn.gelu(jnp.dot(y, w4_ref[...],
                                preferred_element_type=f32) + b4_ref[...])
        out_ref[...] = jnp.dot(y, wo_ref[...], preferred_element_type=f32)


def _k5(h0, agg_pk, batchf3, W_upd, b_upd, W_graph,
        W1, b1, W2, b2, W3, b3, W4, b4, W_out):
    full = lambda shape: pl.BlockSpec(shape, lambda g: tuple(0 for _ in shape))
    return pl.pallas_call(
        _k5_body,
        grid=(G5,),
        in_specs=[
            pl.BlockSpec((NB5, D_NODE), lambda g: (g, 0)),
            pl.BlockSpec((2, NB5, HALF), lambda g: (0, g, 0)),
            pl.BlockSpec((1, 1, NB5), lambda g: (g, 0, 0)),
            full((D_MSG, D_NODE)),
            full((1, D_NODE)),
            full((D_NODE, D_GRAPH)),
            full((D_GRAPH, FC)),
            full((1, FC)),
            full((FC, FC)),
            full((1, FC)),
            full((FC, FC)),
            full((1, FC)),
            full((FC, FC)),
            full((1, FC)),
            full((FC, D_OUT)),
        ],
        out_specs=pl.BlockSpec((NGP, D_OUT), lambda g: (0, 0)),
        out_shape=jax.ShapeDtypeStruct((NGP, D_OUT), f32),
        scratch_shapes=[
            pltpu.VMEM((NGP, D_GRAPH), f32),
            pltpu.VMEM((NGP, NGP), f32),
        ],
    )(h0, agg_pk, batchf3, W_upd, b_upd, W_graph,
      W1, b1, W2, b2, W3, b3, W4, b4, W_out)


def kernel(x, pos, batch, edge_index, W_embed, b_embed, W_msg, W_rbf,
           W_upd, b_upd, W_graph, W1, b1, W2, b2, W3, b3, W4, b4, W_out):
    src_i = edge_index[0].astype(i32)
    dst_i = edge_index[1].astype(i32)
    pos16 = jnp.concatenate([pos, jnp.zeros((N, L - 3), f32)], axis=1)
    batchf3 = batch.astype(f32).reshape(G5, 1, NB5)
    zeros_h = jnp.zeros((N, HALF), f32)

    h0, m_pk = _k1(x, W_embed, b_embed.reshape(1, D_NODE), W_msg)
    diff = _k2(pos16, src_i, dst_i)
    ew_pk = _k3(diff, W_rbf)
    agg_pk = _k4(m_pk, ew_pk, src_i, dst_i, zeros_h)
    out_full = _k5(h0, agg_pk, batchf3, W_upd,
                   b_upd.reshape(1, D_NODE), W_graph,
                   W1, b1.reshape(1, FC), W2, b2.reshape(1, FC),
                   W3, b3.reshape(1, FC), W4, b4.reshape(1, FC), W_out)
    return out_full[:NG]


# speedup vs baseline: 3.5692x; 1.0667x over previous
import functools

import jax
import jax.numpy as jnp
from jax import lax
from jax.experimental import pallas as pl
from jax.experimental.pallas import tpu as pltpu
from jax.experimental.pallas import tpu_sc as plsc

N = 50000
E = 800000
D_IN = 128
D_NODE = 128
D_MSG = 64
HALF = 32
N_RADIAL = 32
D_GRAPH = 256
FC = 256
D_OUT = 128
NG = 100
NGP = 128
CUTOFF = 6.0

NC = 2
NS = 16
L = 16

C = 128

f32 = jnp.float32
i32 = jnp.int32


NB1 = 2000
G1 = N // NB1


def _k1_body(x_ref, we_ref, be_ref, wm_ref, h0_ref, m_ref):
    h = jax.nn.gelu(jnp.dot(x_ref[...], we_ref[...],
                            preferred_element_type=f32) + be_ref[...])
    h0_ref[...] = h
    mm = jnp.dot(h, wm_ref[...], preferred_element_type=f32)
    m_ref[...] = jnp.stack([mm[:, :HALF], mm[:, HALF:]])


def _k1(x, W_embed, b_embed, W_msg):
    return pl.pallas_call(
        _k1_body,
        grid=(G1,),
        in_specs=[
            pl.BlockSpec((NB1, D_IN), lambda g: (g, 0)),
            pl.BlockSpec((D_IN, D_NODE), lambda g: (0, 0)),
            pl.BlockSpec((1, D_NODE), lambda g: (0, 0)),
            pl.BlockSpec((D_NODE, D_MSG), lambda g: (0, 0)),
        ],
        out_specs=[
            pl.BlockSpec((NB1, D_NODE), lambda g: (g, 0)),
            pl.BlockSpec((2, NB1, HALF), lambda g: (0, g, 0)),
        ],
        out_shape=[
            jax.ShapeDtypeStruct((N, D_NODE), f32),
            jax.ShapeDtypeStruct((2, N, HALF), f32),
        ],
    )(x, W_embed, b_embed, W_msg)


EW2 = E // (NC * NS)
N2 = (EW2 + C - 1) // C


def _k2_body(pos_hbm, src_hbm, dst_hbm, diff_hbm,
             is0, id0, ps0, pd0, is1, id1, ps1, pd1, ss0, sd0, ss1, sd1):
    wid = lax.axis_index("s") * NC + lax.axis_index("c")
    base0 = wid * EW2
    bufs = ((is0, id0, ps0, pd0, ss0, sd0),
            (is1, id1, ps1, pd1, ss1, sd1))

    def cbase(ci):
        return base0 + jnp.minimum(ci * C, EW2 - C)

    def fire(ci, p):
        isx, idx, psx, pdx, ss, sd = bufs[p]
        base = cbase(ci)
        pltpu.sync_copy(src_hbm.at[pl.ds(base, C)], isx)
        pltpu.sync_copy(dst_hbm.at[pl.ds(base, C)], idx)
        pltpu.async_copy(pos_hbm.at[isx], psx, ss)
        pltpu.async_copy(pos_hbm.at[idx], pdx, sd)

    def process(ci, p):
        isx, idx, psx, pdx, ss, sd = bufs[p]
        pltpu.make_async_copy(pos_hbm.at[isx], psx, ss).wait()
        pltpu.make_async_copy(pos_hbm.at[idx], pdx, sd).wait()

        def sub8(i, _):
            for u in range(8):
                j = i * 8 + u
                psx[j, :] = psx[j, :] - pdx[j, :]
            return 0
        lax.fori_loop(0, C // 8, sub8, 0)
        pltpu.sync_copy(psx, diff_hbm.at[pl.ds(cbase(ci), C)])

    fire(0, 0)
    fire(1, 1)

    def body2(i2, _):
        i = i2 * 2
        process(i, 0)
        fire(i + 2, 0)
        process(i + 1, 1)
        fire(i + 3, 1)
        return 0
    lax.fori_loop(0, (N2 - 2) // 2, body2, 0)
    process(N2 - 2, 0)
    process(N2 - 1, 1)


def _k2(pos16, src_i, dst_i):
    mesh = plsc.VectorSubcoreMesh(core_axis_name="c", subcore_axis_name="s")
    return pl.kernel(
        _k2_body,
        out_type=jax.ShapeDtypeStruct((E, L), f32),
        mesh=mesh,
        scratch_types=[
            pltpu.VMEM((C,), i32),
            pltpu.VMEM((C,), i32),
            pltpu.VMEM((C, L), f32),
            pltpu.VMEM((C, L), f32),
            pltpu.VMEM((C,), i32),
            pltpu.VMEM((C,), i32),
            pltpu.VMEM((C, L), f32),
            pltpu.VMEM((C, L), f32),
            pltpu.SemaphoreType.DMA,
            pltpu.SemaphoreType.DMA,
            pltpu.SemaphoreType.DMA,
            pltpu.SemaphoreType.DMA,
        ],
        compiler_params=pltpu.CompilerParams(use_tc_tiling_on_sc=False),
    )(pos16, src_i, dst_i)


EB3 = 8000
G3 = E // EB3


def _k3_body(diff_ref, wr_ref, ew_ref):
    t4 = diff_ref[...]
    ssq = jnp.sum(t4 * t4, axis=1, keepdims=True)
    d = jnp.sqrt(ssq + 1e-8)
    mu = lax.broadcasted_iota(i32, (1, N_RADIAL), 1).astype(f32) * (
        CUTOFF / (N_RADIAL - 1))
    sigma = CUTOFF / N_RADIAL
    t = jnp.broadcast_to(d, (EB3, N_RADIAL)) - mu
    rbf = jnp.exp(t * t * (-1.0 / (2.0 * sigma * sigma)))
    wr = wr_ref[...]
    ew0 = jnp.dot(rbf, wr[:, :HALF], preferred_element_type=f32)
    ew1 = jnp.dot(rbf, wr[:, HALF:], preferred_element_type=f32)
    ew_ref[...] = jnp.stack([ew0, ew1])


def _k3(diff, W_rbf):
    return pl.pallas_call(
        _k3_body,
        grid=(G3,),
        in_specs=[
            pl.BlockSpec((EB3, L), lambda g: (g, 0)),
            pl.BlockSpec((N_RADIAL, D_MSG), lambda g: (0, 0)),
        ],
        out_specs=pl.BlockSpec((2, EB3, HALF), lambda g: (0, g, 0)),
        out_shape=jax.ShapeDtypeStruct((2, E, HALF), f32),
    )(diff, W_rbf)


EW4 = E // NS
N4 = EW4 // C
TAIL4 = EW4 - N4 * C
NROW = N // NS


def _k4_body(m_hbm, ew_hbm, src_hbm, dst_hbm, zeros_hbm, agg_hbm,
             acc, is0, id0, rw0, eb0, is1, id1, rw1, eb1,
             sg0, se0, sg1, se1):
    core = lax.axis_index("c")
    sid = lax.axis_index("s")

    pltpu.sync_copy(zeros_hbm.at[pl.ds(sid * NROW, NROW)],
                    acc.at[pl.ds(sid * NROW, NROW)])
    plsc.subcore_barrier()

    base0 = sid * EW4
    bufs = ((is0, id0, rw0, eb0, sg0, se0),
            (is1, id1, rw1, eb1, sg1, se1))

    def fire(c, p, nload):
        isx, idx, rwx, ebx, sg, se = bufs[p]
        base = base0 + c * C
        pltpu.sync_copy(src_hbm.at[pl.ds(base, nload)],
                        isx.at[pl.ds(0, nload)])
        pltpu.sync_copy(dst_hbm.at[pl.ds(base, nload)],
                        idx.at[pl.ds(0, nload)])
        pltpu.async_copy(m_hbm.at[core].at[isx], rwx, sg)
        pltpu.async_copy(ew_hbm.at[core].at[pl.ds(base, nload)],
                        ebx.at[pl.ds(0, nload)], se)

    def process(p, nvalid):
        isx, idx, rwx, ebx, sg, se = bufs[p]
        pltpu.make_async_copy(m_hbm.at[core].at[isx], rwx, sg).wait()
        pltpu.make_async_copy(ew_hbm.at[core].at[pl.ds(0, nvalid)],
                              ebx.at[pl.ds(0, nvalid)], se).wait()

        def mul8(i, _):
            for u in range(8):
                j = i * 8 + u
                for k in range(HALF // L):
                    sl = pl.ds(k * L, L)
                    rwx[j, sl] = rwx[j, sl] * ebx[j, sl]
            return 0
        lax.fori_loop(0, nvalid // 8, mul8, 0)
        if nvalid < C:
            def zero8(i, _):
                for u in range(8):
                    j = nvalid + i * 8 + u
                    for k in range(HALF // L):
                        rwx[j, pl.ds(k * L, L)] = jnp.zeros((L,), f32)
                return 0
            lax.fori_loop(0, (C - nvalid) // 8, zero8, 0)
        pltpu.sync_copy(rwx, acc.at[idx], add=True)

    fire(0, 0, C)
    fire(1, 1, C)

    def body2(i2, _):
        i = i2 * 2
        process(0, C)
        fire(i + 2, 0, C)
        process(1, C)
        fire(i + 3, 1, C)
        return 0
    lax.fori_loop(0, (N4 - 2) // 2, body2, 0)

    process(0, C)
    fire(N4, 0, TAIL4)
    process(1, C)
    process(0, TAIL4)
    plsc.subcore_barrier()

    pltpu.sync_copy(acc.at[pl.ds(sid * NROW, NROW)],
                    agg_hbm.at[core].at[pl.ds(sid * NROW, NROW)])


def _k4(m_pk, ew_pk, src_i, dst_i, zeros_h):
    mesh = plsc.VectorSubcoreMesh(core_axis_name="c", subcore_axis_name="s")
    return pl.kernel(
        _k4_body,
        out_type=jax.ShapeDtypeStruct((2, N, HALF), f32),
        mesh=mesh,
        scratch_types=[
            pltpu.VMEM_SHARED((N, HALF), f32),
            pltpu.VMEM((C,), i32),
            pltpu.VMEM((C,), i32),
            pltpu.VMEM((C, HALF), f32),
            pltpu.VMEM((C, HALF), f32),
            pltpu.VMEM((C,), i32),
            pltpu.VMEM((C,), i32),
            pltpu.VMEM((C, HALF), f32),
            pltpu.VMEM((C, HALF), f32),
            pltpu.SemaphoreType.DMA,
            pltpu.SemaphoreType.DMA,
            pltpu.SemaphoreType.DMA,
            pltpu.SemaphoreType.DMA,
        ],
        compiler_params=pltpu.CompilerParams(use_tc_tiling_on_sc=False),
    )(m_pk, ew_pk, src_i, dst_i, zeros_h)


NB5 = 2000
G5 = N // NB5


def _k5_body(h0_ref, agg_ref, b_ref, wu_ref, bu_ref, wg_ref,
             w1_ref, b1_ref, w2_ref, b2_ref, w3_ref, b3_ref,
             w4_ref, b4_ref, wo_ref, out_ref, pooled, counts):
    g = pl.program_id(0)

    a = jnp.concatenate([agg_ref[0], agg_ref[1]], axis=1)
    upd = jnp.dot(a, wu_ref[...], preferred_element_type=f32) + bu_ref[...]
    h = h0_ref[...] + jax.nn.gelu(upd)
    gg = jnp.dot(h, wg_ref[...], preferred_element_type=f32)

    bb = jnp.broadcast_to(b_ref[0], (NGP, NB5))
    gid = lax.broadcasted_iota(i32, (NGP, NB5), 0).astype(f32)
    oh = jnp.where(bb == gid, 1.0, 0.0).astype(f32)

    @pl.when(g == 0)
    def _():
        pooled[...] = jnp.zeros((NGP, D_GRAPH), f32)
        counts[...] = jnp.zeros((NGP, NGP), f32)

    pooled[...] += jnp.dot(oh, gg, preferred_element_type=f32)
    counts[...] += jnp.broadcast_to(
        jnp.sum(oh, axis=1, keepdims=True), (NGP, NGP))

    @pl.when(g == G5 - 1)
    def _():
        cnt = jnp.maximum(counts[...][:, 0:1], 1.0)
        pm = pooled[...] / cnt
        y = jax.nn.gelu(jnp.dot(pm, w1_ref[...],
                                preferred_element_type=f32) + b1_ref[...])
        y = jax.nn.gelu(jnp.dot(y, w2_ref[...],
                                preferred_element_type=f32) + b2_ref[...])
        y = jax.nn.gelu(jnp.dot(y, w3_ref[...],
                                preferred_element_type=f32) + b3_ref[...])
        y = jax.nn.gelu(jnp.dot(y, w4_ref[...],
                                preferred_element_type=f32) + b4_ref[...])
        out_ref[...] = jnp.dot(y, wo_ref[...], preferred_element_type=f32)


def _k5(h0, agg_pk, batchf3, W_upd, b_upd, W_graph,
        W1, b1, W2, b2, W3, b3, W4, b4, W_out):
    full = lambda shape: pl.BlockSpec(shape, lambda g: tuple(0 for _ in shape))
    return pl.pallas_call(
        _k5_body,
        grid=(G5,),
        in_specs=[
            pl.BlockSpec((NB5, D_NODE), lambda g: (g, 0)),
            pl.BlockSpec((2, NB5, HALF), lambda g: (0, g, 0)),
            pl.BlockSpec((1, 1, NB5), lambda g: (g, 0, 0)),
            full((D_MSG, D_NODE)),
            full((1, D_NODE)),
            full((D_NODE, D_GRAPH)),
            full((D_GRAPH, FC)),
            full((1, FC)),
            full((FC, FC)),
            full((1, FC)),
            full((FC, FC)),
            full((1, FC)),
            full((FC, FC)),
            full((1, FC)),
            full((FC, D_OUT)),
        ],
        out_specs=pl.BlockSpec((NGP, D_OUT), lambda g: (0, 0)),
        out_shape=jax.ShapeDtypeStruct((NGP, D_OUT), f32),
        scratch_shapes=[
            pltpu.VMEM((NGP, D_GRAPH), f32),
            pltpu.VMEM((NGP, NGP), f32),
        ],
    )(h0, agg_pk, batchf3, W_upd, b_upd, W_graph,
      W1, b1, W2, b2, W3, b3, W4, b4, W_out)


def kernel(x, pos, batch, edge_index, W_embed, b_embed, W_msg, W_rbf,
           W_upd, b_upd, W_graph, W1, b1, W2, b2, W3, b3, W4, b4, W_out):
    src_i = edge_index[0].astype(i32)
    dst_i = edge_index[1].astype(i32)
    pos16 = jnp.concatenate([pos, jnp.zeros((N, L - 3), f32)], axis=1)
    batchf3 = batch.astype(f32).reshape(G5, 1, NB5)
    zeros_h = jnp.zeros((N, HALF), f32)

    h0, m_pk = _k1(x, W_embed, b_embed.reshape(1, D_NODE), W_msg)
    diff = _k2(pos16, src_i, dst_i)
    ew_pk = _k3(diff, W_rbf)
    agg_pk = _k4(m_pk, ew_pk, src_i, dst_i, zeros_h)
    out_full = _k5(h0, agg_pk, batchf3, W_upd,
                   b_upd.reshape(1, D_NODE), W_graph,
                   W1, b1.reshape(1, FC), W2, b2.reshape(1, FC),
                   W3, b3.reshape(1, FC), W4, b4.reshape(1, FC), W_out)
    return out_full[:NG]
